# baseline (speedup 1.0000x reference)
import functools

import jax
import jax.numpy as jnp
from jax import lax
from jax.experimental import pallas as pl
from jax.experimental.pallas import tpu as pltpu
from jax.experimental.pallas import tpu_sc as plsc

_NC, _NS = 2, 16
_CH = 128


def _sc_scatter_add(table, src, dst, init):
    e_total = src.shape[0]
    n_out, feat = init.shape
    if n_out * feat * 4 <= 7 * 2 ** 20:
        acc2 = _sc_scatter_full(table, src, dst, n_out, feat)
        return _halves_sum(acc2, init)
    return _sc_scatter_split(table, src, dst, init)


def _sc_scatter_full(table, src, dst, n_out, feat):
    e_total = src.shape[0]
    per_w = e_total // (_NC * _NS)
    n_chunks = per_w // _CH
    rps = n_out // _NS
    assert per_w % _CH == 0 and n_out % (_NS * _CH) == 0

    def body(table_h, src_h, dst_h, zero_h, out_h, dstv, rows, acc, gsem):
        cid = lax.axis_index("c")
        sid = lax.axis_index("s")
        row0 = pl.multiple_of(sid * rps, 8)
        pltpu.sync_copy(zero_h.at[pl.ds(row0, rps)], acc.at[pl.ds(row0, rps)])
        plsc.subcore_barrier()

        wid = sid * _NC + cid
        e0 = wid * per_w

        def step(k, carry):
            off = pl.multiple_of(e0 + k * _CH, 8)
            pltpu.sync_copy(src_h.at[pl.ds(off, _CH)], dstv)
            pltpu.async_copy(table_h.at[dstv], rows, gsem).wait()
            pltpu.sync_copy(dst_h.at[pl.ds(off, _CH)], dstv)
            pltpu.sync_copy(rows, acc.at[dstv], add=True)
            return carry

        lax.fori_loop(0, n_chunks, step, 0)
        plsc.subcore_barrier()
        base = pl.multiple_of(cid * n_out, 8)
        pltpu.sync_copy(acc.at[pl.ds(row0, rps)],
                        out_h.at[pl.ds(base + row0, rps)])

    fn = pl.kernel(
        body,
        out_type=jax.ShapeDtypeStruct((2 * n_out, feat), jnp.float32),
        mesh=plsc.VectorSubcoreMesh(core_axis_name="c", subcore_axis_name="s"),
        scratch_types=[
            pltpu.VMEM((_CH,), jnp.int32),
            pltpu.VMEM((_CH, feat), jnp.float32),
            pltpu.VMEM_SHARED((n_out, feat), jnp.float32),
            pltpu.SemaphoreType.DMA,
        ],
        compiler_params=pltpu.CompilerParams(use_tc_tiling_on_sc=False),
    )
    return fn(table, src, dst, jnp.zeros((n_out, feat), jnp.float32))


def _halves_sum(acc2, init):
    n, feat = init.shape
    br = 1024

    def body(a_ref, b_ref, i_ref, o_ref):
        o_ref[...] = a_ref[...] + b_ref[...] + i_ref[...]

    nb = n // br
    return pl.pallas_call(
        body,
        grid=(nb,),
        in_specs=[
            pl.BlockSpec((br, feat), lambda i: (i, 0)),
            pl.BlockSpec((br, feat), lambda i, _n=nb: (i + _n, 0)),
            pl.BlockSpec((br, feat), lambda i: (i, 0)),
        ],
        out_specs=pl.BlockSpec((br, feat), lambda i: (i, 0)),
        out_shape=jax.ShapeDtypeStruct((n, feat), jnp.float32),
    )(acc2, acc2, init)


def _sc_scatter_split(table, src, dst, init):
    e_total = src.shape[0]
    n_out, feat = init.shape
    n_half = n_out // _NC
    per_sub = e_total // _NS
    n_chunks = per_sub // _CH
    rps = n_half // _NS
    assert per_sub % _CH == 0 and n_out % _NC == 0 and n_half % _NS == 0

    def body(table_h, src_h, dst_h, init_h, out_h, srcv, dstv, locv, rows,
             acc, gsem):
        cid = lax.axis_index("c")
        sid = lax.axis_index("s")
        base = pl.multiple_of(cid * n_half, 8)
        row0 = pl.multiple_of(sid * rps, 8)
        pltpu.sync_copy(init_h.at[pl.ds(base + row0, rps)],
                        acc.at[pl.ds(row0, rps)])
        plsc.subcore_barrier()

        e0 = sid * per_sub

        def step(k, carry):
            off = pl.multiple_of(e0 + k * _CH, 8)
            pltpu.sync_copy(src_h.at[pl.ds(off, _CH)], srcv)
            pltpu.sync_copy(dst_h.at[pl.ds(off, _CH)], dstv)
            for j in range(_CH // 16):
                v = dstv[pl.ds(j * 16, 16)]
                loc = v - base
                oob = (loc < 0) | (loc >= n_half)
                locv[pl.ds(j * 16, 16)] = jnp.where(oob, n_half, loc)
            pltpu.async_copy(table_h.at[srcv], rows, gsem).wait()
            pltpu.sync_copy(rows, acc.at[locv], add=True)
            return carry

        lax.fori_loop(0, n_chunks, step, 0)
        plsc.subcore_barrier()
        pltpu.sync_copy(acc.at[pl.ds(row0, rps)],
                        out_h.at[pl.ds(base + row0, rps)])

    fn = pl.kernel(
        body,
        out_type=jax.ShapeDtypeStruct((n_out, feat), jnp.float32),
        mesh=plsc.VectorSubcoreMesh(core_axis_name="c", subcore_axis_name="s"),
        scratch_types=[
            pltpu.VMEM((_CH,), jnp.int32),
            pltpu.VMEM((_CH,), jnp.int32),
            pltpu.VMEM((_CH,), jnp.int32),
            pltpu.VMEM((_CH, feat), jnp.float32),
            pltpu.VMEM_SHARED((n_half + 8, feat), jnp.float32),
            pltpu.SemaphoreType.DMA,
        ],
        compiler_params=pltpu.CompilerParams(use_tc_tiling_on_sc=False),
    )
    return fn(table, src, dst, init)


def _gcn_pre(x, deg16, b8, w, first):
    n, fin = x.shape
    fout = w.shape[1]
    br = 1024

    def body(x_ref, d_ref, b_ref, w_ref, o_ref):
        dinv = lax.rsqrt(d_ref[:, 0:1])
        if first:
            a = x_ref[...]
        else:
            a = jnp.maximum(x_ref[...] * dinv + b_ref[0:1, :], 0.0)
        o_ref[...] = jnp.dot(a, w_ref[...],
                             preferred_element_type=jnp.float32) * dinv

    return pl.pallas_call(
        body,
        grid=(n // br,),
        in_specs=[
            pl.BlockSpec((br, fin), lambda i: (i, 0)),
            pl.BlockSpec((br, 16), lambda i: (i, 0)),
            pl.BlockSpec((8, fin), lambda i: (0, 0)),
            pl.BlockSpec((fin, fout), lambda i: (0, 0)),
        ],
        out_specs=pl.BlockSpec((br, fout), lambda i: (i, 0)),
        out_shape=jax.ShapeDtypeStruct((n, fout), jnp.float32),
    )(x, deg16, b8, w)


def _sage_mid(agg, a, wl, bl8, wr, act):
    n, fin = a.shape
    fout = wl.shape[1]
    br = 2048

    def body(g_ref, a_ref, wl_ref, b_ref, wr_ref, o_ref):
        z = jnp.dot(g_ref[...], wl_ref[...], preferred_element_type=jnp.float32)
        z = z + jnp.dot(a_ref[...], wr_ref[...],
                        preferred_element_type=jnp.float32)
        z = z + b_ref[0:1, :]
        o_ref[...] = jnp.maximum(z, 0.0) if act else z

    return pl.pallas_call(
        body,
        grid=(n // br,),
        in_specs=[
            pl.BlockSpec((br, fin), lambda i: (i, 0)),
            pl.BlockSpec((br, fin), lambda i: (i, 0)),
            pl.BlockSpec((fin, fout), lambda i: (0, 0)),
            pl.BlockSpec((8, fout), lambda i: (0, 0)),
            pl.BlockSpec((fin, fout), lambda i: (0, 0)),
        ],
        out_specs=pl.BlockSpec((br, fout), lambda i: (i, 0)),
        out_shape=jax.ShapeDtypeStruct((n, fout), jnp.float32),
    )(agg, a, wl, bl8, wr)


def _pool_mol(acc3, deg3, b8):
    ng = acc3.shape[0]
    bb = 32

    def body(a_ref, d_ref, b_ref, o_ref):
        dinv = lax.rsqrt(d_ref[:, :, 0:1])
        z = a_ref[...] * dinv + b_ref[0:1, :][None, :, :]
        o_ref[...] = jnp.max(z, axis=1)

    return pl.pallas_call(
        body,
        grid=(ng // bb,),
        in_specs=[
            pl.BlockSpec((bb, 32, 128), lambda i: (i, 0, 0)),
            pl.BlockSpec((bb, 32, 16), lambda i: (i, 0, 0)),
            pl.BlockSpec((8, 128), lambda i: (0, 0)),
        ],
        out_specs=pl.BlockSpec((bb, 128), lambda i: (i, 0)),
        out_shape=jax.ShapeDtypeStruct((ng, 128), jnp.float32),
    )(acc3, deg3, b8)


def _pool_tgt(z3):
    ng = z3.shape[0]
    bb = 32

    def body(z_ref, o_ref):
        o_ref[...] = jnp.sum(z_ref[...], axis=1) * (1.0 / z3.shape[1])

    return pl.pallas_call(
        body,
        grid=(ng // bb,),
        in_specs=[pl.BlockSpec((bb, z3.shape[1], 128), lambda i: (i, 0, 0))],
        out_specs=pl.BlockSpec((bb, 128), lambda i: (i, 0)),
        out_shape=jax.ShapeDtypeStruct((ng, 128), jnp.float32),
    )(z3)


def _split16(x):
    xh = x.astype(jnp.bfloat16)
    xl = (x - xh.astype(jnp.float32)).astype(jnp.bfloat16)
    return xh, xl


def _dot3(xh, xl, wh, wl, dims):
    def d(a, b):
        return lax.dot_general(a, b, dims,
                               preferred_element_type=jnp.float32)
    return d(xh, wh) + d(xh, wl) + d(xl, wh)


def _leaky(x):
    return jnp.where(x > 0, x, 0.01 * x)


def _conv_stage1(x3, k1p, b1, ls, lv1, bb):
    nb = x3.shape[0]
    c1 = 32
    lc = ls - 8

    def body(x_ref, k_ref, b_ref, y_ref, s_ref, acc_ref):
        i = pl.program_id(0)
        kw = k_ref[...]
        acc_ref[...] = jnp.zeros((bb, lc, c1), jnp.float32)
        for t in range(3):
            acc_ref[...] += lax.dot_general(
                x_ref[:, t:t + lc, :], kw[t], (((2,), (0,)), ((), ())),
                preferred_element_type=jnp.float32)
        acc = acc_ref[...] + b_ref[0:1, :][None, :, :]
        lmask = lax.broadcasted_iota(jnp.int32, (bb, lc, c1), 1) < lv1
        acc = jnp.where(lmask, acc, 0.0)
        y_ref[...] = jnp.zeros((bb, ls, c1), jnp.float32)
        y_ref[:, :lc, :] = acc

        @pl.when(i == 0)
        def _():
            s_ref[...] = jnp.zeros((8, c1), jnp.float32)

        s_ref[0:1, :] += jnp.sum(acc, axis=(0, 1))[None, :]
        s_ref[1:2, :] += jnp.sum(acc * acc, axis=(0, 1))[None, :]

    return pl.pallas_call(
        body,
        grid=(nb // bb,),
        in_specs=[
            pl.BlockSpec((bb, ls, 8), lambda i: (i, 0, 0)),
            pl.BlockSpec((8, 8, c1), lambda i: (0, 0, 0)),
            pl.BlockSpec((8, c1), lambda i: (0, 0)),
        ],
        out_specs=[
            pl.BlockSpec((bb, ls, c1), lambda i: (i, 0, 0)),
            pl.BlockSpec((8, c1), lambda i: (0, 0)),
        ],
        out_shape=[
            jax.ShapeDtypeStruct((nb, ls, c1), jnp.float32),
            jax.ShapeDtypeStruct((8, c1), jnp.float32),
        ],
        scratch_shapes=[pltpu.VMEM((bb, lc, c1), jnp.float32)],
    )(x3, k1p, b1)


def _bn_fold(s_ref, g_ref, beta_ref, cnt):
    m = s_ref[0:1, :] * (1.0 / cnt)
    var = s_ref[1:2, :] * (1.0 / cnt) - m * m
    a = g_ref[0:1, :] * lax.rsqrt(var + 1e-5)
    return a, beta_ref[0:1, :] - m * a


def _conv_stage2(y1, s1, k2, b2, g1, beta1, cin, cout, ktaps, ls, lv_in,
                 lv_out, cnt, bb):
    nb = y1.shape[0]
    lc = ls - 8

    def body(y_ref, s_ref, k_ref, b_ref, g_ref, bt_ref, o_ref, so_ref, acc_ref):
        i = pl.program_id(0)
        a, bc = _bn_fold(s_ref, g_ref, bt_ref, cnt)
        z = _leaky(y_ref[...] * a[None, :, :] + bc[None, :, :])
        zmask = lax.broadcasted_iota(jnp.int32, (bb, ls, cin), 1) < lv_in
        z = jnp.where(zmask, z, 0.0)
        kw = k_ref[...]
        acc_ref[...] = jnp.zeros((bb, lc, cout), jnp.float32)
        for t in range(ktaps):
            acc_ref[...] += lax.dot_general(
                z[:, t:t + lc, :], kw[t], (((2,), (0,)), ((), ())),
                preferred_element_type=jnp.float32)
        acc = acc_ref[...] + b_ref[0:1, :][None, :, :]
        lmask = lax.broadcasted_iota(jnp.int32, (bb, lc, cout), 1) < lv_out
        acc = jnp.where(lmask, acc, 0.0)
        o_ref[...] = jnp.zeros((bb, ls, cout), jnp.float32)
        o_ref[:, :lc, :] = acc

        @pl.when(i == 0)
        def _():
            so_ref[...] = jnp.zeros((8, cout), jnp.float32)

        so_ref[0:1, :] += jnp.sum(acc, axis=(0, 1))[None, :]
        so_ref[1:2, :] += jnp.sum(acc * acc, axis=(0, 1))[None, :]

    return pl.pallas_call(
        body,
        grid=(nb // bb,),
        in_specs=[
            pl.BlockSpec((bb, ls, cin), lambda i: (i, 0, 0)),
            pl.BlockSpec((8, cin), lambda i: (0, 0)),
            pl.BlockSpec((8, cin, cout), lambda i: (0, 0, 0)),
            pl.BlockSpec((8, cout), lambda i: (0, 0)),
            pl.BlockSpec((8, cin), lambda i: (0, 0)),
            pl.BlockSpec((8, cin), lambda i: (0, 0)),
        ],
        out_specs=[
            pl.BlockSpec((bb, ls, cout), lambda i: (i, 0, 0)),
            pl.BlockSpec((8, cout), lambda i: (0, 0)),
        ],
        out_shape=[
            jax.ShapeDtypeStruct((nb, ls, cout), jnp.float32),
            jax.ShapeDtypeStruct((8, cout), jnp.float32),
        ],
        scratch_shapes=[pltpu.VMEM((bb, lc, cout), jnp.float32)],
    )(y1, s1, k2, b2, g1, beta1)


def _conv_stage3(y2, s2, k3, b3, g2, beta2, cin, cout, ktaps, ls, lv_in,
                 lv_out, cnt, bb):
    nb = y2.shape[0]
    lc = ls - 8

    def body(y_ref, s_ref, k_ref, b_ref, g_ref, bt_ref, o_ref, acc_ref):
        a, bc = _bn_fold(s_ref, g_ref, bt_ref, cnt)
        z = _leaky(y_ref[...] * a[None, :, :] + bc[None, :, :])
        zmask = lax.broadcasted_iota(jnp.int32, (bb, ls, cin), 1) < lv_in
        z = jnp.where(zmask, z, 0.0)
        kw = k_ref[...]
        acc_ref[...] = jnp.zeros((bb, lc, cout), jnp.float32)
        for t in range(ktaps):
            acc_ref[...] += lax.dot_general(
                z[:, t:t + lc, :], kw[t], (((2,), (0,)), ((), ())),
                preferred_element_type=jnp.float32)
        acc = acc_ref[...] + b_ref[0:1, :][None, :, :]
        lmask = lax.broadcasted_iota(jnp.int32, (bb, lc, cout), 1) < lv_out
        acc = jnp.where(lmask, acc, -1e30)
        o_ref[...] = jnp.max(acc, axis=1)

    return pl.pallas_call(
        body,
        grid=(nb // bb,),
        in_specs=[
            pl.BlockSpec((bb, ls, cin), lambda i: (i, 0, 0)),
            pl.BlockSpec((8, cin), lambda i: (0, 0)),
            pl.BlockSpec((8, cin, cout), lambda i: (0, 0, 0)),
            pl.BlockSpec((8, cout), lambda i: (0, 0)),
            pl.BlockSpec((8, cin), lambda i: (0, 0)),
            pl.BlockSpec((8, cin), lambda i: (0, 0)),
        ],
        out_specs=pl.BlockSpec((bb, cout), lambda i: (i, 0)),
        out_shape=jax.ShapeDtypeStruct((nb, cout), jnp.float32),
        scratch_shapes=[pltpu.VMEM((bb, lc, cout), jnp.float32)],
    )(y2, s2, k3, b3, g2, beta2)


def _tower(x, p, pre, lin, ls, bbs):
    lv1, lv2, lv3 = lin - 2, lin - 6, lin - 12
    nb = x.shape[0]
    x3 = jnp.pad(x[:, :, None], ((0, 0), (0, ls - lin), (0, 7)))
    k1p = jnp.pad(jnp.transpose(p[pre + '_k'][0], (2, 1, 0)),
                  ((0, 5), (0, 7), (0, 0)))
    k2 = jnp.pad(jnp.transpose(p[pre + '_k'][1], (2, 1, 0)),
                 ((0, 3), (0, 0), (0, 0)))
    k3 = jnp.pad(jnp.transpose(p[pre + '_k'][2], (2, 1, 0)),
                 ((0, 1), (0, 0), (0, 0)))

    def b8(v):
        return jnp.pad(v[None, :], ((0, 7), (0, 0)))

    y1, s1 = _conv_stage1(x3, k1p, b8(p[pre + '_b'][0]), ls, lv1, bbs[0])
    y2, s2 = _conv_stage2(y1, s1, k2, b8(p[pre + '_b'][1]),
                          b8(p[pre + '_g'][0]), b8(p[pre + '_beta'][0]),
                          32, 64, 5, ls, lv1, lv2, nb * lv1, bbs[1])
    return _conv_stage3(y2, s2, k3, b8(p[pre + '_b'][2]),
                        b8(p[pre + '_g'][1]), b8(p[pre + '_beta'][1]),
                        64, 128, 7, ls, lv2, lv3, nb * lv2, bbs[2])


def _supcon_pair(f1, f2, temperature=0.5):
    f = jnp.concatenate([f1, f2], axis=0)
    f = f * lax.rsqrt(jnp.sum(f * f, axis=1, keepdims=True))
    bsz = f.shape[0]
    b = f1.shape[0]
    adc = jnp.dot(f, f.T, preferred_element_type=jnp.float32) / temperature
    logits = adc - jnp.max(adc, axis=1, keepdims=True)
    expl = jnp.exp(logits)
    r = lax.broadcasted_iota(jnp.int32, (bsz, bsz), 0)
    c = lax.broadcasted_iota(jnp.int32, (bsz, bsz), 1)
    eye = (r == c)
    mask = (r % b) == (c % b)
    posf = jnp.where(mask & (~eye), 1.0, 0.0)
    keep = jnp.where(eye, 0.0, 1.0)
    denom = jnp.sum(expl * keep, axis=1, keepdims=True)
    lp = logits - jnp.log(denom)
    lps = jnp.sum(lp * posf, axis=1) / jnp.sum(posf, axis=1)
    return -jnp.mean(lps) * temperature


def _supcon(xm, drug, xt, prot):
    def body(a_ref, b_ref, c_ref, d_ref, o_ref):
        total = (_supcon_pair(a_ref[...], b_ref[...]) +
                 _supcon_pair(c_ref[...], d_ref[...]))
        o_ref[...] = jnp.zeros((8, 128), jnp.float32)
        o_ref[0:1, 0:1] = total[None, None]

    return pl.pallas_call(
        body,
        out_shape=jax.ShapeDtypeStruct((8, 128), jnp.float32),
    )(xm, drug, xt, prot)


def _head(xm, drug, xt, prot, hw, fc1w, fc1b, fc2w, fc2b, outw, outb):
    def body(xm_ref, dg_ref, xt_ref, pt_ref, g0w, g0b, n0w, n0b, l0w, l0b,
             g1w, g1b, n1w, n1b, l1w, l1b, w1, b1, w2, b2, w3, b3, o_ref):
        h = jnp.concatenate(
            [xm_ref[...], dg_ref[...], xt_ref[...], pt_ref[...]], axis=1)
        for gw, gb, nw, nb2, lw, lb in ((g0w, g0b, n0w, n0b, l0w, l0b),
                                        (g1w, g1b, n1w, n1b, l1w, l1b)):
            gz = jnp.dot(h, gw[...]) + gb[0:1, :]
            g = 1.0 / (1.0 + jnp.exp(-gz))
            nl = jnp.maximum(jnp.dot(h, nw[...]) + nb2[0:1, :], 0.0)
            li = jnp.dot(h, lw[...]) + lb[0:1, :]
            h = g * nl + (1.0 - g) * li
        xc = _leaky(jnp.dot(h, w1[...]) + b1[0:1, :])
        xc = _leaky(jnp.dot(xc, w2[...]) + b2[0:1, :])
        o_ref[...] = jnp.dot(xc, w3[...]) + b3[0:1, :]

    args = [xm, drug, xt, prot] + hw + [fc1w, fc1b, fc2w, fc2b, outw, outb]
    return pl.pallas_call(
        body,
        out_shape=jax.ShapeDtypeStruct((xm.shape[0], 8), jnp.float32),
    )(*args)


def kernel(mol_x, mol_edge_index, mol_batch, target_x, target_edge_index,
           target_batch, smiles_emb, fasta_emb, params):
    p = params
    nm = mol_x.shape[0]
    nt = target_x.shape[0]
    ng = smiles_emb.shape[0]

    def b8(v, n):
        return jnp.pad(v[None, :], ((0, 7), (0, n - v.shape[0])))

    src, dst = mol_edge_index[0], mol_edge_index[1]
    ones16 = jnp.ones((nm, 16), jnp.float32)
    deg16 = _sc_scatter_add(ones16, src, dst, ones16)

    molx = jnp.pad(mol_x, ((0, 0), (0, 2)))
    w0 = jnp.pad(p['gcn_W'][0], ((0, 2), (0, 2)))
    w1 = jnp.pad(p['gcn_W'][1], ((0, 2), (0, 4)))
    w2 = jnp.pad(p['gcn_W'][2], ((0, 4), (0, 0)))
    b0 = b8(p['gcn_b'][0], 80)
    b1 = b8(p['gcn_b'][1], 160)
    b2 = b8(p['gcn_b'][2], 128)

    hs0 = _gcn_pre(molx, deg16, b0, w0, True)
    acc0 = _sc_scatter_add(hs0, src, dst, hs0)
    hs1 = _gcn_pre(acc0, deg16, b0, w1, False)
    acc1 = _sc_scatter_add(hs1, src, dst, hs1)
    hs2 = _gcn_pre(acc1, deg16, b1, w2, False)
    acc2 = _sc_scatter_add(hs2, src, dst, hs2)
    xm = _pool_mol(acc2.reshape(ng, nm // ng, 128),
                   deg16.reshape(ng, nm // ng, 16), b2)

    tsrc, tdst = target_edge_index[0], target_edge_index[1]
    a0 = jnp.pad(target_x, ((0, 0), (0, 7)))
    wl0 = jnp.pad(p['sage_Wl'][0], ((0, 7), (0, 7)))
    wl1 = jnp.pad(p['sage_Wl'][1], ((0, 7), (0, 6)))
    wl2 = jnp.pad(p['sage_Wl'][2], ((0, 6), (0, 0)))
    wr0 = jnp.pad(p['sage_Wr'][0], ((0, 7), (0, 7)))
    wr1 = jnp.pad(p['sage_Wr'][1], ((0, 7), (0, 6)))
    wr2 = jnp.pad(p['sage_Wr'][2], ((0, 6), (0, 0)))
    bl0 = b8(p['sage_bl'][0], 40)
    bl1 = b8(p['sage_bl'][1], 72)
    bl2 = b8(p['sage_bl'][2], 128)

    agg0 = _sc_scatter_add(a0, tsrc, tdst, jnp.zeros((nt, 40), jnp.float32))
    a1 = _sage_mid(agg0, a0, wl0, bl0, wr0, True)
    agg1 = _sc_scatter_add(a1, tsrc, tdst, jnp.zeros((nt, 40), jnp.float32))
    a2 = _sage_mid(agg1, a1, wl1, bl1, wr1, True)
    agg2 = jnp.concatenate(
        [_sc_scatter_add(a2[:, :40], tsrc, tdst,
                         jnp.zeros((nt, 40), jnp.float32)),
         _sc_scatter_add(a2[:, 40:], tsrc, tdst,
                         jnp.zeros((nt, 32), jnp.float32))], axis=1)
    z2 = _sage_mid(agg2, a2, wl2, bl2, wr2, False)
    xt = _pool_tgt(z2.reshape(ng, nt // ng, 128))

    drug = _tower(smiles_emb, p, 'd', 100, 112, (32, 64, 64))
    prot = _tower(fasta_emb, p, 'p', 1200, 1208, (8, 8, 8))

    con8 = _supcon(xm, drug, xt, prot)

    hw = []
    for l in range(2):
        hw += [p['hw_gW'][l], b8(p['hw_gb'][l], 512),
               p['hw_nW'][l], b8(p['hw_nb'][l], 512),
               p['hw_lW'][l], b8(p['hw_lb'][l], 512)]
    out8 = _head(xm, drug, xt, prot, hw,
                 p['fc1_W'], b8(p['fc1_b'], 1024),
                 p['fc2_W'], b8(p['fc2_b'], 512),
                 jnp.pad(p['out_W'], ((0, 0), (0, 7))), b8(p['out_b'], 8))

    return (out8[:, 0:1], con8[0, 0])


# speedup vs baseline: 1.1032x; 1.1032x over previous
import functools

import jax
import jax.numpy as jnp
from jax import lax
from jax.experimental import pallas as pl
from jax.experimental.pallas import tpu as pltpu
from jax.experimental.pallas import tpu_sc as plsc

_NC, _NS = 2, 16
_CH = 128


def _sc_scatter_add(table, src, dst, init):
    e_total = src.shape[0]
    n_out, feat = init.shape
    if n_out * feat * 4 <= 7 * 2 ** 20:
        acc2 = _sc_scatter_full(table, src, dst, n_out, feat)
        return _halves_sum(acc2, init)
    return _sc_scatter_split(table, src, dst, init)


def _sc_scatter_full(table, src, dst, n_out, feat):
    e_total = src.shape[0]
    per_w = e_total // (_NC * _NS)
    n_chunks = per_w // _CH
    rps = n_out // _NS
    assert per_w % _CH == 0 and n_out % (_NS * _CH) == 0

    def body(table_h, src_h, dst_h, zero_h, out_h, dstv, rows, acc, gsem):
        cid = lax.axis_index("c")
        sid = lax.axis_index("s")
        row0 = pl.multiple_of(sid * rps, 8)
        pltpu.sync_copy(zero_h.at[pl.ds(row0, rps)], acc.at[pl.ds(row0, rps)])
        plsc.subcore_barrier()

        wid = sid * _NC + cid
        e0 = wid * per_w

        def step(k, carry):
            off = pl.multiple_of(e0 + k * _CH, 8)
            pltpu.sync_copy(src_h.at[pl.ds(off, _CH)], dstv)
            pltpu.async_copy(table_h.at[dstv], rows, gsem).wait()
            pltpu.sync_copy(dst_h.at[pl.ds(off, _CH)], dstv)
            pltpu.sync_copy(rows, acc.at[dstv], add=True)
            return carry

        lax.fori_loop(0, n_chunks, step, 0)
        plsc.subcore_barrier()
        base = pl.multiple_of(cid * n_out, 8)
        pltpu.sync_copy(acc.at[pl.ds(row0, rps)],
                        out_h.at[pl.ds(base + row0, rps)])

    fn = pl.kernel(
        body,
        out_type=jax.ShapeDtypeStruct((2 * n_out, feat), jnp.float32),
        mesh=plsc.VectorSubcoreMesh(core_axis_name="c", subcore_axis_name="s"),
        scratch_types=[
            pltpu.VMEM((_CH,), jnp.int32),
            pltpu.VMEM((_CH, feat), jnp.float32),
            pltpu.VMEM_SHARED((n_out, feat), jnp.float32),
            pltpu.SemaphoreType.DMA,
        ],
        compiler_params=pltpu.CompilerParams(use_tc_tiling_on_sc=False),
    )
    return fn(table, src, dst, jnp.zeros((n_out, feat), jnp.float32))


def _halves_sum(acc2, init):
    n, feat = init.shape
    br = 1024

    def body(a_ref, b_ref, i_ref, o_ref):
        o_ref[...] = a_ref[...] + b_ref[...] + i_ref[...]

    nb = n // br
    return pl.pallas_call(
        body,
        grid=(nb,),
        in_specs=[
            pl.BlockSpec((br, feat), lambda i: (i, 0)),
            pl.BlockSpec((br, feat), lambda i, _n=nb: (i + _n, 0)),
            pl.BlockSpec((br, feat), lambda i: (i, 0)),
        ],
        out_specs=pl.BlockSpec((br, feat), lambda i: (i, 0)),
        out_shape=jax.ShapeDtypeStruct((n, feat), jnp.float32),
    )(acc2, acc2, init)


def _sc_scatter_split(table, src, dst, init):
    e_total = src.shape[0]
    n_out, feat = init.shape
    n_half = n_out // _NC
    per_sub = e_total // _NS
    n_chunks = per_sub // _CH
    rps = n_half // _NS
    assert per_sub % _CH == 0 and n_out % _NC == 0 and n_half % _NS == 0

    def body(table_h, src_h, dst_h, init_h, out_h, srcv, dstv, locv, rows,
             acc, gsem):
        cid = lax.axis_index("c")
        sid = lax.axis_index("s")
        base = pl.multiple_of(cid * n_half, 8)
        row0 = pl.multiple_of(sid * rps, 8)
        pltpu.sync_copy(init_h.at[pl.ds(base + row0, rps)],
                        acc.at[pl.ds(row0, rps)])
        plsc.subcore_barrier()

        e0 = sid * per_sub

        def step(k, carry):
            off = pl.multiple_of(e0 + k * _CH, 8)
            pltpu.sync_copy(src_h.at[pl.ds(off, _CH)], srcv)
            pltpu.sync_copy(dst_h.at[pl.ds(off, _CH)], dstv)
            for j in range(_CH // 16):
                v = dstv[pl.ds(j * 16, 16)]
                loc = v - base
                oob = (loc < 0) | (loc >= n_half)
                locv[pl.ds(j * 16, 16)] = jnp.where(oob, n_half, loc)
            pltpu.async_copy(table_h.at[srcv], rows, gsem).wait()
            pltpu.sync_copy(rows, acc.at[locv], add=True)
            return carry

        lax.fori_loop(0, n_chunks, step, 0)
        plsc.subcore_barrier()
        pltpu.sync_copy(acc.at[pl.ds(row0, rps)],
                        out_h.at[pl.ds(base + row0, rps)])

    fn = pl.kernel(
        body,
        out_type=jax.ShapeDtypeStruct((n_out, feat), jnp.float32),
        mesh=plsc.VectorSubcoreMesh(core_axis_name="c", subcore_axis_name="s"),
        scratch_types=[
            pltpu.VMEM((_CH,), jnp.int32),
            pltpu.VMEM((_CH,), jnp.int32),
            pltpu.VMEM((_CH,), jnp.int32),
            pltpu.VMEM((_CH, feat), jnp.float32),
            pltpu.VMEM_SHARED((n_half + 8, feat), jnp.float32),
            pltpu.SemaphoreType.DMA,
        ],
        compiler_params=pltpu.CompilerParams(use_tc_tiling_on_sc=False),
    )
    return fn(table, src, dst, init)


def _gcn_pre(x, deg16, b8, w, first):
    n, fin = x.shape
    fout = w.shape[1]
    br = 1024

    def body(x_ref, d_ref, b_ref, w_ref, o_ref):
        dinv = lax.rsqrt(d_ref[:, 0:1])
        if first:
            a = x_ref[...]
        else:
            a = jnp.maximum(x_ref[...] * dinv + b_ref[0:1, :], 0.0)
        o_ref[...] = jnp.dot(a, w_ref[...],
                             preferred_element_type=jnp.float32) * dinv

    return pl.pallas_call(
        body,
        grid=(n // br,),
        in_specs=[
            pl.BlockSpec((br, fin), lambda i: (i, 0)),
            pl.BlockSpec((br, 16), lambda i: (i, 0)),
            pl.BlockSpec((8, fin), lambda i: (0, 0)),
            pl.BlockSpec((fin, fout), lambda i: (0, 0)),
        ],
        out_specs=pl.BlockSpec((br, fout), lambda i: (i, 0)),
        out_shape=jax.ShapeDtypeStruct((n, fout), jnp.float32),
    )(x, deg16, b8, w)


def _sage_mid(agg, a, wl, bl8, wr, act):
    n, fin = a.shape
    fout = wl.shape[1]
    br = 2048

    def body(g_ref, a_ref, wl_ref, b_ref, wr_ref, o_ref):
        z = jnp.dot(g_ref[...], wl_ref[...], preferred_element_type=jnp.float32)
        z = z + jnp.dot(a_ref[...], wr_ref[...],
                        preferred_element_type=jnp.float32)
        z = z + b_ref[0:1, :]
        o_ref[...] = jnp.maximum(z, 0.0) if act else z

    return pl.pallas_call(
        body,
        grid=(n // br,),
        in_specs=[
            pl.BlockSpec((br, fin), lambda i: (i, 0)),
            pl.BlockSpec((br, fin), lambda i: (i, 0)),
            pl.BlockSpec((fin, fout), lambda i: (0, 0)),
            pl.BlockSpec((8, fout), lambda i: (0, 0)),
            pl.BlockSpec((fin, fout), lambda i: (0, 0)),
        ],
        out_specs=pl.BlockSpec((br, fout), lambda i: (i, 0)),
        out_shape=jax.ShapeDtypeStruct((n, fout), jnp.float32),
    )(agg, a, wl, bl8, wr)


def _pool_mol(acc3, deg3, b8):
    ng = acc3.shape[0]
    bb = 32

    def body(a_ref, d_ref, b_ref, o_ref):
        dinv = lax.rsqrt(d_ref[:, :, 0:1])
        z = a_ref[...] * dinv + b_ref[0:1, :][None, :, :]
        o_ref[...] = jnp.max(z, axis=1)

    return pl.pallas_call(
        body,
        grid=(ng // bb,),
        in_specs=[
            pl.BlockSpec((bb, 32, 128), lambda i: (i, 0, 0)),
            pl.BlockSpec((bb, 32, 16), lambda i: (i, 0, 0)),
            pl.BlockSpec((8, 128), lambda i: (0, 0)),
        ],
        out_specs=pl.BlockSpec((bb, 128), lambda i: (i, 0)),
        out_shape=jax.ShapeDtypeStruct((ng, 128), jnp.float32),
    )(acc3, deg3, b8)


def _pool_tgt(z3):
    ng = z3.shape[0]
    bb = 32

    def body(z_ref, o_ref):
        o_ref[...] = jnp.sum(z_ref[...], axis=1) * (1.0 / z3.shape[1])

    return pl.pallas_call(
        body,
        grid=(ng // bb,),
        in_specs=[pl.BlockSpec((bb, z3.shape[1], 128), lambda i: (i, 0, 0))],
        out_specs=pl.BlockSpec((bb, 128), lambda i: (i, 0)),
        out_shape=jax.ShapeDtypeStruct((ng, 128), jnp.float32),
    )(z3)


def _split16(x):
    xh = x.astype(jnp.bfloat16)
    xl = (x - xh.astype(jnp.float32)).astype(jnp.bfloat16)
    return xh, xl


def _dot3(xh, xl, wh, wl, dims):
    def d(a, b):
        return lax.dot_general(a, b, dims,
                               preferred_element_type=jnp.float32)
    return d(xh, wh) + d(xh, wl) + d(xl, wh)


def _leaky(x):
    return jnp.where(x > 0, x, 0.01 * x)


def _conv_stage1(x3, k1p, b1, ls, lv1, bb):
    nb = x3.shape[0]
    c1 = 32
    lc = ls - 8

    def body(x_ref, k_ref, b_ref, y_ref, s_ref, acc_ref):
        i = pl.program_id(0)
        kw = k_ref[...]
        acc_ref[...] = jnp.zeros((bb, lc, c1), jnp.float32)
        for t in range(3):
            acc_ref[...] += lax.dot_general(
                x_ref[:, t:t + lc, :], kw[t], (((2,), (0,)), ((), ())),
                preferred_element_type=jnp.float32)
        acc = acc_ref[...] + b_ref[0:1, :][None, :, :]
        lmask = lax.broadcasted_iota(jnp.int32, (bb, lc, c1), 1) < lv1
        acc = jnp.where(lmask, acc, 0.0)
        y_ref[...] = jnp.zeros((bb, ls, c1), jnp.float32)
        y_ref[:, :lc, :] = acc

        @pl.when(i == 0)
        def _():
            s_ref[...] = jnp.zeros((8, c1), jnp.float32)

        s_ref[0:1, :] += jnp.sum(acc, axis=(0, 1))[None, :]
        s_ref[1:2, :] += jnp.sum(acc * acc, axis=(0, 1))[None, :]

    return pl.pallas_call(
        body,
        grid=(nb // bb,),
        in_specs=[
            pl.BlockSpec((bb, ls, 8), lambda i: (i, 0, 0)),
            pl.BlockSpec((8, 8, c1), lambda i: (0, 0, 0)),
            pl.BlockSpec((8, c1), lambda i: (0, 0)),
        ],
        out_specs=[
            pl.BlockSpec((bb, ls, c1), lambda i: (i, 0, 0)),
            pl.BlockSpec((8, c1), lambda i: (0, 0)),
        ],
        out_shape=[
            jax.ShapeDtypeStruct((nb, ls, c1), jnp.float32),
            jax.ShapeDtypeStruct((8, c1), jnp.float32),
        ],
        scratch_shapes=[pltpu.VMEM((bb, lc, c1), jnp.float32)],
    )(x3, k1p, b1)


def _bn_fold(s_ref, g_ref, beta_ref, cnt):
    m = s_ref[0:1, :] * (1.0 / cnt)
    var = s_ref[1:2, :] * (1.0 / cnt) - m * m
    a = g_ref[0:1, :] * lax.rsqrt(var + 1e-5)
    return a, beta_ref[0:1, :] - m * a


def _conv_stage2(y1, s1, k2, b2, g1, beta1, cin, cout, ktaps, ls, lv_in,
                 lv_out, cnt, bb):
    nb = y1.shape[0]
    lc = ls - 8

    def body(y_ref, s_ref, k_ref, b_ref, g_ref, bt_ref, o_ref, so_ref, acc_ref):
        i = pl.program_id(0)
        a, bc = _bn_fold(s_ref, g_ref, bt_ref, cnt)
        z = _leaky(y_ref[...] * a[None, :, :] + bc[None, :, :])
        zmask = lax.broadcasted_iota(jnp.int32, (bb, ls, cin), 1) < lv_in
        z = jnp.where(zmask, z, 0.0)
        kw = k_ref[...]
        acc_ref[...] = jnp.zeros((bb, lc, cout), jnp.float32)
        for t in range(ktaps):
            acc_ref[...] += lax.dot_general(
                z[:, t:t + lc, :], kw[t], (((2,), (0,)), ((), ())),
                preferred_element_type=jnp.float32)
        acc = acc_ref[...] + b_ref[0:1, :][None, :, :]
        lmask = lax.broadcasted_iota(jnp.int32, (bb, lc, cout), 1) < lv_out
        acc = jnp.where(lmask, acc, 0.0)
        o_ref[...] = jnp.zeros((bb, ls, cout), jnp.float32)
        o_ref[:, :lc, :] = acc

        @pl.when(i == 0)
        def _():
            so_ref[...] = jnp.zeros((8, cout), jnp.float32)

        so_ref[0:1, :] += jnp.sum(acc, axis=(0, 1))[None, :]
        so_ref[1:2, :] += jnp.sum(acc * acc, axis=(0, 1))[None, :]

    return pl.pallas_call(
        body,
        grid=(nb // bb,),
        in_specs=[
            pl.BlockSpec((bb, ls, cin), lambda i: (i, 0, 0)),
            pl.BlockSpec((8, cin), lambda i: (0, 0)),
            pl.BlockSpec((8, cin, cout), lambda i: (0, 0, 0)),
            pl.BlockSpec((8, cout), lambda i: (0, 0)),
            pl.BlockSpec((8, cin), lambda i: (0, 0)),
            pl.BlockSpec((8, cin), lambda i: (0, 0)),
        ],
        out_specs=[
            pl.BlockSpec((bb, ls, cout), lambda i: (i, 0, 0)),
            pl.BlockSpec((8, cout), lambda i: (0, 0)),
        ],
        out_shape=[
            jax.ShapeDtypeStruct((nb, ls, cout), jnp.float32),
            jax.ShapeDtypeStruct((8, cout), jnp.float32),
        ],
        scratch_shapes=[pltpu.VMEM((bb, lc, cout), jnp.float32)],
    )(y1, s1, k2, b2, g1, beta1)


def _conv_stage3(y2, s2, k3, b3, g2, beta2, cin, cout, ktaps, ls, lv_in,
                 lv_out, cnt, bb):
    nb = y2.shape[0]
    lc = ls - 8

    def body(y_ref, s_ref, k_ref, b_ref, g_ref, bt_ref, o_ref, acc_ref):
        a, bc = _bn_fold(s_ref, g_ref, bt_ref, cnt)
        z = _leaky(y_ref[...] * a[None, :, :] + bc[None, :, :])
        zmask = lax.broadcasted_iota(jnp.int32, (bb, ls, cin), 1) < lv_in
        z = jnp.where(zmask, z, 0.0)
        kw = k_ref[...]
        acc_ref[...] = jnp.zeros((bb, lc, cout), jnp.float32)
        for t in range(ktaps):
            acc_ref[...] += lax.dot_general(
                z[:, t:t + lc, :], kw[t], (((2,), (0,)), ((), ())),
                preferred_element_type=jnp.float32)
        acc = acc_ref[...] + b_ref[0:1, :][None, :, :]
        lmask = lax.broadcasted_iota(jnp.int32, (bb, lc, cout), 1) < lv_out
        acc = jnp.where(lmask, acc, -1e30)
        o_ref[...] = jnp.max(acc, axis=1)

    return pl.pallas_call(
        body,
        grid=(nb // bb,),
        in_specs=[
            pl.BlockSpec((bb, ls, cin), lambda i: (i, 0, 0)),
            pl.BlockSpec((8, cin), lambda i: (0, 0)),
            pl.BlockSpec((8, cin, cout), lambda i: (0, 0, 0)),
            pl.BlockSpec((8, cout), lambda i: (0, 0)),
            pl.BlockSpec((8, cin), lambda i: (0, 0)),
            pl.BlockSpec((8, cin), lambda i: (0, 0)),
        ],
        out_specs=pl.BlockSpec((bb, cout), lambda i: (i, 0)),
        out_shape=jax.ShapeDtypeStruct((nb, cout), jnp.float32),
        scratch_shapes=[pltpu.VMEM((bb, lc, cout), jnp.float32)],
    )(y2, s2, k3, b3, g2, beta2)


def _tower(x, p, pre, lin, ls, bbs):
    lv1, lv2, lv3 = lin - 2, lin - 6, lin - 12
    nb = x.shape[0]
    x3 = jnp.pad(x[:, :, None], ((0, 0), (0, ls - lin), (0, 7)))
    k1p = jnp.pad(jnp.transpose(p[pre + '_k'][0], (2, 1, 0)),
                  ((0, 5), (0, 7), (0, 0)))
    k2 = jnp.pad(jnp.transpose(p[pre + '_k'][1], (2, 1, 0)),
                 ((0, 3), (0, 0), (0, 0)))
    k3 = jnp.pad(jnp.transpose(p[pre + '_k'][2], (2, 1, 0)),
                 ((0, 1), (0, 0), (0, 0)))

    def b8(v):
        return jnp.pad(v[None, :], ((0, 7), (0, 0)))

    y1, s1 = _conv_stage1(x3, k1p, b8(p[pre + '_b'][0]), ls, lv1, bbs[0])
    y2, s2 = _conv_stage2(y1, s1, k2, b8(p[pre + '_b'][1]),
                          b8(p[pre + '_g'][0]), b8(p[pre + '_beta'][0]),
                          32, 64, 5, ls, lv1, lv2, nb * lv1, bbs[1])
    return _conv_stage3(y2, s2, k3, b8(p[pre + '_b'][2]),
                        b8(p[pre + '_g'][1]), b8(p[pre + '_beta'][1]),
                        64, 128, 7, ls, lv2, lv3, nb * lv2, bbs[2])


def _supcon_pair(f1, f2, temperature=0.5):
    f = jnp.concatenate([f1, f2], axis=0)
    f = f * lax.rsqrt(jnp.sum(f * f, axis=1, keepdims=True))
    bsz = f.shape[0]
    b = f1.shape[0]
    adc = jnp.dot(f, f.T, preferred_element_type=jnp.float32) / temperature
    logits = adc - jnp.max(adc, axis=1, keepdims=True)
    expl = jnp.exp(logits)
    r = lax.broadcasted_iota(jnp.int32, (bsz, bsz), 0)
    c = lax.broadcasted_iota(jnp.int32, (bsz, bsz), 1)
    eye = (r == c)
    mask = (r % b) == (c % b)
    posf = jnp.where(mask & (~eye), 1.0, 0.0)
    keep = jnp.where(eye, 0.0, 1.0)
    denom = jnp.sum(expl * keep, axis=1, keepdims=True)
    lp = logits - jnp.log(denom)
    lps = jnp.sum(lp * posf, axis=1) / jnp.sum(posf, axis=1)
    return -jnp.mean(lps) * temperature


def _supcon(xm, drug, xt, prot):
    def body(a_ref, b_ref, c_ref, d_ref, o_ref):
        total = (_supcon_pair(a_ref[...], b_ref[...]) +
                 _supcon_pair(c_ref[...], d_ref[...]))
        o_ref[...] = jnp.zeros((8, 128), jnp.float32)
        o_ref[0:1, 0:1] = total[None, None]

    return pl.pallas_call(
        body,
        out_shape=jax.ShapeDtypeStruct((8, 128), jnp.float32),
    )(xm, drug, xt, prot)


def _head(xm, drug, xt, prot, hw, fc1w, fc1b, fc2w, fc2b, outw, outb):
    def body(xm_ref, dg_ref, xt_ref, pt_ref, g0w, g0b, n0w, n0b, l0w, l0b,
             g1w, g1b, n1w, n1b, l1w, l1b, w1, b1, w2, b2, w3, b3, o_ref):
        h = jnp.concatenate(
            [xm_ref[...], dg_ref[...], xt_ref[...], pt_ref[...]], axis=1)
        for gw, gb, nw, nb2, lw, lb in ((g0w, g0b, n0w, n0b, l0w, l0b),
                                        (g1w, g1b, n1w, n1b, l1w, l1b)):
            gz = jnp.dot(h, gw[...]) + gb[0:1, :]
            g = 1.0 / (1.0 + jnp.exp(-gz))
            nl = jnp.maximum(jnp.dot(h, nw[...]) + nb2[0:1, :], 0.0)
            li = jnp.dot(h, lw[...]) + lb[0:1, :]
            h = g * nl + (1.0 - g) * li
        xc = _leaky(jnp.dot(h, w1[...]) + b1[0:1, :])
        xc = _leaky(jnp.dot(xc, w2[...]) + b2[0:1, :])
        o_ref[...] = jnp.dot(xc, w3[...]) + b3[0:1, :]

    args = [xm, drug, xt, prot] + hw + [fc1w, fc1b, fc2w, fc2b, outw, outb]
    return pl.pallas_call(
        body,
        out_shape=jax.ShapeDtypeStruct((xm.shape[0], 8), jnp.float32),
    )(*args)


def kernel(mol_x, mol_edge_index, mol_batch, target_x, target_edge_index,
           target_batch, smiles_emb, fasta_emb, params):
    p = params
    nm = mol_x.shape[0]
    nt = target_x.shape[0]
    ng = smiles_emb.shape[0]

    def b8(v, n):
        return jnp.pad(v[None, :], ((0, 7), (0, n - v.shape[0])))

    src, dst = mol_edge_index[0], mol_edge_index[1]
    ones16 = jnp.ones((nm, 16), jnp.float32)
    deg16 = _sc_scatter_add(ones16, src, dst, ones16)

    molx = jnp.pad(mol_x, ((0, 0), (0, 2)))
    w0 = jnp.pad(p['gcn_W'][0], ((0, 2), (0, 2)))
    w1 = jnp.pad(p['gcn_W'][1], ((0, 2), (0, 4)))
    w2 = jnp.pad(p['gcn_W'][2], ((0, 4), (0, 0)))
    b0 = b8(p['gcn_b'][0], 80)
    b1 = b8(p['gcn_b'][1], 160)
    b2 = b8(p['gcn_b'][2], 128)

    hs0 = _gcn_pre(molx, deg16, b0, w0, True)
    acc0 = _sc_scatter_add(hs0, src, dst, hs0)
    hs1 = _gcn_pre(acc0, deg16, b0, w1, False)
    acc1 = _sc_scatter_add(hs1, src, dst, hs1)
    hs2 = _gcn_pre(acc1, deg16, b1, w2, False)
    acc2 = _sc_scatter_add(hs2, src, dst, hs2)
    xm = _pool_mol(acc2.reshape(ng, nm // ng, 128),
                   deg16.reshape(ng, nm // ng, 16), b2)

    tsrc, tdst = target_edge_index[0], target_edge_index[1]
    a0 = jnp.pad(target_x, ((0, 0), (0, 7)))
    wl0 = jnp.pad(p['sage_Wl'][0], ((0, 7), (0, 7)))
    wl1 = jnp.pad(p['sage_Wl'][1], ((0, 7), (0, 6)))
    wl2 = jnp.pad(p['sage_Wl'][2], ((0, 6), (0, 0)))
    wr0 = jnp.pad(p['sage_Wr'][0], ((0, 7), (0, 7)))
    wr1 = jnp.pad(p['sage_Wr'][1], ((0, 7), (0, 6)))
    wr2 = jnp.pad(p['sage_Wr'][2], ((0, 6), (0, 0)))
    bl0 = b8(p['sage_bl'][0], 40)
    bl1 = b8(p['sage_bl'][1], 72)
    bl2 = b8(p['sage_bl'][2], 128)

    agg0 = _sc_scatter_add(a0, tsrc, tdst, jnp.zeros((nt, 40), jnp.float32))
    a1 = _sage_mid(agg0, a0, wl0, bl0, wr0, True)
    agg1 = _sc_scatter_add(a1, tsrc, tdst, jnp.zeros((nt, 40), jnp.float32))
    a2 = _sage_mid(agg1, a1, wl1, bl1, wr1, True)
    agg2 = _sc_scatter_add(a2, tsrc, tdst, jnp.zeros((nt, 72), jnp.float32))
    z2 = _sage_mid(agg2, a2, wl2, bl2, wr2, False)
    xt = _pool_tgt(z2.reshape(ng, nt // ng, 128))

    drug = _tower(smiles_emb, p, 'd', 100, 112, (32, 64, 64))
    prot = _tower(fasta_emb, p, 'p', 1200, 1208, (8, 8, 8))

    con8 = _supcon(xm, drug, xt, prot)

    hw = []
    for l in range(2):
        hw += [p['hw_gW'][l], b8(p['hw_gb'][l], 512),
               p['hw_nW'][l], b8(p['hw_nb'][l], 512),
               p['hw_lW'][l], b8(p['hw_lb'][l], 512)]
    out8 = _head(xm, drug, xt, prot, hw,
                 p['fc1_W'], b8(p['fc1_b'], 1024),
                 p['fc2_W'], b8(p['fc2_b'], 512),
                 jnp.pad(p['out_W'], ((0, 0), (0, 7))), b8(p['out_b'], 8))

    return (out8[:, 0:1], con8[0, 0])
